# baseline (speedup 1.0000x reference)
import functools

import jax
import jax.numpy as jnp
from jax import lax
from jax.experimental import pallas as pl
from jax.experimental.pallas import tpu as pltpu
from jax.experimental.pallas import tpu_sc as plsc

_SUB = 128
_SPG = 4


@functools.lru_cache(maxsize=None)
def _make_gather(n_rows, d):
    info = plsc.get_sparse_core_info()
    nw = info.num_cores * info.num_subcores
    nc = info.num_cores
    assert n_rows % (nw * _SPG * _SUB) == 0
    groups_per_w = n_rows // (nw * _SPG * _SUB)
    n_subs = n_rows // _SUB

    mesh = plsc.VectorSubcoreMesh(core_axis_name="c", subcore_axis_name="s")

    @functools.partial(
        pl.kernel,
        mesh=mesh,
        out_type=jax.ShapeDtypeStruct((n_subs, _SUB, d), jnp.float32),
        compiler_params=pltpu.CompilerParams(use_tc_tiling_on_sc=False),
        scratch_types=[
            pltpu.VMEM((_SPG, _SUB), jnp.int32),
            pltpu.VMEM((_SPG, _SUB, d), jnp.float32),
            pltpu.SemaphoreType.DMA,
        ],
    )
    def gather_kernel(table_hbm, idx_hbm, out_hbm, idx_v, rows_v, gsem):
        wid = lax.axis_index("s") * nc + lax.axis_index("c")
        g0 = wid * groups_per_w

        def body(g, carry):
            grp = g0 + g
            pltpu.sync_copy(idx_hbm.at[pl.ds(grp * _SPG, _SPG)], idx_v)
            descs = [
                pltpu.async_copy(table_hbm.at[idx_v.at[j]], rows_v.at[j], gsem)
                for j in range(_SPG)
            ]
            for dsc in descs:
                dsc.wait()
            pltpu.sync_copy(rows_v, out_hbm.at[pl.ds(grp * _SPG, _SPG)])
            return carry

        lax.fori_loop(0, groups_per_w, body, 0)

    return gather_kernel


def kernel(indices, tables):
    f, v, d = tables.shape
    b = indices.shape[0]
    n_rows = b * f
    flat_idx = (
        indices.astype(jnp.int32) + (jnp.arange(f, dtype=jnp.int32) * v)[None, :]
    ).reshape(n_rows // _SUB, _SUB)
    table2d = tables.reshape(f * v, d)
    out = _make_gather(n_rows, d)(table2d, flat_idx)
    return out.reshape(b, f, d)


# speedup vs baseline: 1.0126x; 1.0126x over previous
import functools

import jax
import jax.numpy as jnp
from jax import lax
from jax.experimental import pallas as pl
from jax.experimental.pallas import tpu as pltpu
from jax.experimental.pallas import tpu_sc as plsc

_SUB = 128
_SPG = 4


@functools.lru_cache(maxsize=None)
def _make_gather(n_rows, d):
    info = plsc.get_sparse_core_info()
    nw = info.num_cores * info.num_subcores
    nc = info.num_cores
    assert n_rows % (nw * _SPG * _SUB) == 0
    groups_per_w = n_rows // (nw * _SPG * _SUB)
    assert groups_per_w % 2 == 0 and groups_per_w >= 4
    n_subs = n_rows // _SUB

    mesh = plsc.VectorSubcoreMesh(core_axis_name="c", subcore_axis_name="s")

    @functools.partial(
        pl.kernel,
        mesh=mesh,
        out_type=jax.ShapeDtypeStruct((n_subs, _SUB, d), jnp.float32),
        compiler_params=pltpu.CompilerParams(use_tc_tiling_on_sc=False),
        scratch_types=[
            pltpu.VMEM((2, _SPG, _SUB), jnp.int32),
            pltpu.VMEM((2, _SPG, _SUB, d), jnp.float32),
            pltpu.SemaphoreType.DMA,
            pltpu.SemaphoreType.DMA,
            pltpu.SemaphoreType.DMA,
            pltpu.SemaphoreType.DMA,
            pltpu.SemaphoreType.DMA,
            pltpu.SemaphoreType.DMA,
        ],
    )
    def gather_kernel(table_hbm, idx_hbm, out_hbm, idx_v, rows_v,
                      isem0, isem1, gsem0, gsem1, osem0, osem1):
        isem = (isem0, isem1)
        gsem = (gsem0, gsem1)
        osem = (osem0, osem1)
        wid = lax.axis_index("s") * nc + lax.axis_index("c")
        g0 = wid * groups_per_w

        def fire_idx(grp, p):
            pltpu.async_copy(
                idx_hbm.at[pl.ds(grp * _SPG, _SPG)], idx_v.at[p], isem[p])

        def wait_idx(p):
            pltpu.make_async_copy(
                idx_hbm.at[pl.ds(0, _SPG)], idx_v.at[p], isem[p]).wait()

        def fire_gather(p):
            for j in range(_SPG):
                pltpu.async_copy(
                    table_hbm.at[idx_v.at[p, j]], rows_v.at[p, j], gsem[p])

        def wait_gather(p):
            pltpu.make_async_copy(
                out_hbm.at[pl.ds(0, _SPG)], rows_v.at[p], gsem[p]).wait()

        def fire_write(grp, p):
            pltpu.async_copy(
                rows_v.at[p], out_hbm.at[pl.ds(grp * _SPG, _SPG)], osem[p])

        def wait_write(p):
            pltpu.make_async_copy(
                rows_v.at[p], out_hbm.at[pl.ds(0, _SPG)], osem[p]).wait()

        fire_idx(g0, 0)
        wait_idx(0)
        fire_gather(0)
        fire_idx(g0 + 1, 1)
        wait_gather(0)
        fire_write(g0, 0)
        wait_idx(1)
        fire_gather(1)
        fire_idx(g0 + 2, 0)

        def body(u, carry):
            t0 = 2 * u
            wait_gather(1)
            fire_write(g0 + t0 - 1, 1)
            wait_write(0)
            wait_idx(0)
            fire_gather(0)
            fire_idx(g0 + (t0 + 1) % groups_per_w, 1)
            wait_gather(0)
            fire_write(g0 + t0, 0)
            wait_write(1)
            wait_idx(1)
            fire_gather(1)
            fire_idx(g0 + (t0 + 2) % groups_per_w, 0)
            return carry

        lax.fori_loop(1, groups_per_w // 2, body, 0)

        wait_gather(1)
        fire_write(g0 + groups_per_w - 1, 1)
        wait_write(0)
        wait_write(1)
        wait_idx(0)

    return gather_kernel


def kernel(indices, tables):
    f, v, d = tables.shape
    b = indices.shape[0]
    n_rows = b * f
    flat_idx = (
        indices.astype(jnp.int32) + (jnp.arange(f, dtype=jnp.int32) * v)[None, :]
    ).reshape(n_rows // _SUB, _SUB)
    table2d = tables.reshape(f * v, d)
    out = _make_gather(n_rows, d)(table2d, flat_idx)
    return out.reshape(b, f, d)


# speedup vs baseline: 1.0143x; 1.0017x over previous
import functools

import jax
import jax.numpy as jnp
from jax import lax
from jax.experimental import pallas as pl
from jax.experimental.pallas import tpu as pltpu
from jax.experimental.pallas import tpu_sc as plsc

_GRP = 512


@functools.lru_cache(maxsize=None)
def _make_gather(n_rows, d):
    info = plsc.get_sparse_core_info()
    nw = info.num_cores * info.num_subcores
    nc = info.num_cores
    assert n_rows % (nw * _GRP) == 0
    groups_per_w = n_rows // (nw * _GRP)
    assert groups_per_w % 2 == 0 and groups_per_w >= 4
    n_groups = n_rows // _GRP

    mesh = plsc.VectorSubcoreMesh(core_axis_name="c", subcore_axis_name="s")

    @functools.partial(
        pl.kernel,
        mesh=mesh,
        out_type=jax.ShapeDtypeStruct((n_groups, _GRP, d), jnp.float32),
        compiler_params=pltpu.CompilerParams(use_tc_tiling_on_sc=False),
        scratch_types=[
            pltpu.VMEM((2, _GRP), jnp.int32),
            pltpu.VMEM((2, _GRP, d), jnp.float32),
            pltpu.SemaphoreType.DMA,
            pltpu.SemaphoreType.DMA,
            pltpu.SemaphoreType.DMA,
            pltpu.SemaphoreType.DMA,
            pltpu.SemaphoreType.DMA,
            pltpu.SemaphoreType.DMA,
        ],
    )
    def gather_kernel(table_hbm, idx_hbm, out_hbm, idx_v, rows_v,
                      isem0, isem1, gsem0, gsem1, osem0, osem1):
        isem = (isem0, isem1)
        gsem = (gsem0, gsem1)
        osem = (osem0, osem1)
        wid = lax.axis_index("s") * nc + lax.axis_index("c")
        g0 = wid * groups_per_w

        def fire_idx(grp, p):
            pltpu.async_copy(idx_hbm.at[grp], idx_v.at[p], isem[p])

        def wait_idx(p):
            pltpu.make_async_copy(idx_hbm.at[0], idx_v.at[p], isem[p]).wait()

        def fire_gather(p):
            pltpu.async_copy(table_hbm.at[idx_v.at[p]], rows_v.at[p], gsem[p])

        def wait_gather(p):
            pltpu.make_async_copy(out_hbm.at[0], rows_v.at[p], gsem[p]).wait()

        def fire_write(grp, p):
            pltpu.async_copy(rows_v.at[p], out_hbm.at[grp], osem[p])

        def wait_write(p):
            pltpu.make_async_copy(rows_v.at[p], out_hbm.at[0], osem[p]).wait()

        fire_idx(g0, 0)
        wait_idx(0)
        fire_gather(0)
        fire_idx(g0 + 1, 1)
        wait_gather(0)
        fire_write(g0, 0)
        wait_idx(1)
        fire_gather(1)
        fire_idx(g0 + 2, 0)

        def body(u, carry):
            t0 = 2 * u
            wait_gather(1)
            fire_write(g0 + t0 - 1, 1)
            wait_write(0)
            wait_idx(0)
            fire_gather(0)
            fire_idx(g0 + (t0 + 1) % groups_per_w, 1)
            wait_gather(0)
            fire_write(g0 + t0, 0)
            wait_write(1)
            wait_idx(1)
            fire_gather(1)
            fire_idx(g0 + (t0 + 2) % groups_per_w, 0)
            return carry

        lax.fori_loop(1, groups_per_w // 2, body, 0)

        wait_gather(1)
        fire_write(g0 + groups_per_w - 1, 1)
        wait_write(0)
        wait_write(1)
        wait_idx(0)

    return gather_kernel


def kernel(indices, tables):
    f, v, d = tables.shape
    b = indices.shape[0]
    n_rows = b * f
    flat_idx = (
        indices.astype(jnp.int32) + (jnp.arange(f, dtype=jnp.int32) * v)[None, :]
    ).reshape(n_rows // _GRP, _GRP)
    table2d = tables.reshape(f * v, d)
    out = _make_gather(n_rows, d)(table2d, flat_idx)
    return out.reshape(b, f, d)
